# baseline (speedup 1.0000x reference)
import math

import jax
import jax.numpy as jnp
from jax.experimental import pallas as pl
from jax.experimental.pallas import tpu as pltpu

MAX_L = 8
K = (MAX_L + 1) ** 2
ROWS_PER_BLOCK = 256


def _sph_cols(x, y, z):
    r = jnp.sqrt(x * x + y * y + z * z)
    ct = jnp.clip(z / r, -1.0, 1.0)
    st = jnp.sqrt(jnp.maximum(1.0 - ct * ct, 0.0))
    rho = jnp.sqrt(x * x + y * y)
    safe = rho > 0.0
    inv_rho = 1.0 / jnp.where(safe, rho, 1.0)
    ca = jnp.where(safe, x * inv_rho, 1.0)
    sa = jnp.where(safe, y * inv_rho, 0.0)

    cm = {1: ca}
    sm = {1: sa}
    for m in range(1, MAX_L):
        cm[m + 1] = cm[m] * ca - sm[m] * sa
        sm[m + 1] = sm[m] * ca + cm[m] * sa

    P = {(0, 0): jnp.ones_like(ct)}
    for m in range(1, MAX_L + 1):
        P[(m, m)] = -(2 * m - 1) * st * P[(m - 1, m - 1)]
    for m in range(0, MAX_L):
        P[(m + 1, m)] = (2 * m + 1) * ct * P[(m, m)]
    for m in range(0, MAX_L + 1):
        for l in range(m + 2, MAX_L + 1):
            P[(l, m)] = ((2 * l - 1) * ct * P[(l - 1, m)]
                         - (l + m - 1) * P[(l - 2, m)]) / (l - m)

    cols = [None] * K
    sqrt2 = math.sqrt(2.0)
    for l in range(MAX_L + 1):
        for m in range(-l, l + 1):
            am = abs(m)
            n = math.sqrt((2 * l + 1) / (4.0 * math.pi)
                          * math.factorial(l - am) / math.factorial(l + am))
            base = n * P[(l, am)]
            if m > 0:
                y_lm = (sqrt2 * base) * cm[m]
            elif m == 0:
                y_lm = base
            else:
                y_lm = (sqrt2 * base) * sm[am]
            cols[l * (l + 1) + m] = y_lm
    return cols


def _tph_kernel(c_ref, o_ref):
    c = c_ref[...]
    cols1 = _sph_cols(c[:, 0:1], c[:, 1:2], c[:, 2:3])
    cols2 = _sph_cols(c[:, 3:4], c[:, 4:5], c[:, 5:6])
    psi2 = jnp.concatenate(cols2, axis=1)
    for i in range(K):
        o_ref[:, i * K:(i + 1) * K] = cols1[i] * psi2


def _tph_call(coordinates, interpret=False):
    n = coordinates.shape[0]
    r = ROWS_PER_BLOCK
    return pl.pallas_call(
        _tph_kernel,
        grid=(n // r,),
        in_specs=[pl.BlockSpec((r, 6), lambda i: (i, 0))],
        out_specs=pl.BlockSpec((r, K * K), lambda i: (i, 0)),
        out_shape=jax.ShapeDtypeStruct((n, K * K), jnp.float32),
        compiler_params=pltpu.CompilerParams(
            dimension_semantics=("parallel",),
            vmem_limit_bytes=56 * 1024 * 1024,
        ),
        interpret=interpret,
    )(coordinates)


@jax.jit
def kernel(coordinates):
    return _tph_call(coordinates)


# speedup vs baseline: 1.5260x; 1.5260x over previous
import math

import jax
import jax.numpy as jnp
from jax.experimental import pallas as pl
from jax.experimental.pallas import tpu as pltpu

MAX_L = 8
K = (MAX_L + 1) ** 2
ROWS_PER_BLOCK = 256
LANES = 128


def _sph_cols(x, y, z):
    rho2 = x * x + y * y
    r2 = rho2 + z * z
    inv_r = jax.lax.rsqrt(r2)
    ct = jnp.clip(z * inv_r, -1.0, 1.0)
    st = jnp.sqrt(jnp.maximum(1.0 - ct * ct, 0.0))
    safe = rho2 > 0.0
    inv_rho = jax.lax.rsqrt(jnp.where(safe, rho2, 1.0))
    ca = jnp.where(safe, x * inv_rho, 1.0)
    sa = jnp.where(safe, y * inv_rho, 0.0)

    sqrt2 = math.sqrt(2.0)
    cm = {1: ca}
    sm = {1: sa}
    for m in range(1, MAX_L):
        cm[m + 1] = cm[m] * ca - sm[m] * sa
        sm[m + 1] = sm[m] * ca + cm[m] * sa
    c2 = {m: sqrt2 * cm[m] for m in cm}
    s2 = {m: sqrt2 * sm[m] for m in sm}

    P = {(0, 0): jnp.full_like(ct, math.sqrt(1.0 / (4.0 * math.pi)))}
    for m in range(1, MAX_L + 1):
        c = -math.sqrt((2 * m + 1) / (2.0 * m))
        P[(m, m)] = (c * st) * P[(m - 1, m - 1)]
    for m in range(0, MAX_L):
        c = math.sqrt(2 * m + 3)
        P[(m + 1, m)] = (c * ct) * P[(m, m)]
    for m in range(0, MAX_L + 1):
        for l in range(m + 2, MAX_L + 1):
            a = math.sqrt((4.0 * l * l - 1.0) / (l * l - m * m))
            b = -math.sqrt(((2 * l + 1.0) * ((l - 1) ** 2 - m * m))
                           / ((2 * l - 3.0) * (l * l - m * m)))
            P[(l, m)] = a * ct * P[(l - 1, m)] + b * P[(l - 2, m)]

    cols = [None] * K
    for l in range(MAX_L + 1):
        for m in range(-l, l + 1):
            am = abs(m)
            if m > 0:
                y_lm = P[(l, am)] * c2[m]
            elif m == 0:
                y_lm = P[(l, 0)]
            else:
                y_lm = P[(l, am)] * s2[am]
            cols[l * (l + 1) + m] = y_lm
    return cols


def _tph_kernel(c_ref, o_ref):
    c = c_ref[...]
    rep = [jnp.broadcast_to(c[:, k:k + 1], (ROWS_PER_BLOCK, LANES))
           for k in range(6)]
    cols1 = _sph_cols(rep[0], rep[1], rep[2])
    cols2 = _sph_cols(rep[3], rep[4], rep[5])

    lane = jax.lax.broadcasted_iota(jnp.int32, (ROWS_PER_BLOCK, K), 1)
    psi2 = cols2[K - 1][:, :K]
    for j in range(K - 2, -1, -1):
        psi2 = jnp.where(lane == j, cols2[j][:, :K], psi2)

    for i in range(K):
        o_ref[:, i * K:(i + 1) * K] = cols1[i][:, :K] * psi2


def _tph_call(coordinates, interpret=False):
    n = coordinates.shape[0]
    r = ROWS_PER_BLOCK
    return pl.pallas_call(
        _tph_kernel,
        grid=(n // r,),
        in_specs=[pl.BlockSpec((r, 6), lambda i: (i, 0))],
        out_specs=pl.BlockSpec((r, K * K), lambda i: (i, 0)),
        out_shape=jax.ShapeDtypeStruct((n, K * K), jnp.float32),
        compiler_params=pltpu.CompilerParams(
            dimension_semantics=("parallel",),
            vmem_limit_bytes=56 * 1024 * 1024,
        ),
        interpret=interpret,
    )(coordinates)


@jax.jit
def kernel(coordinates):
    return _tph_call(coordinates)
